# Initial kernel scaffold; baseline (speedup 1.0000x reference)
import jax
import jax.numpy as jnp
from jax.experimental import pallas as pl


def kernel(x, gate_W, gate_b, W1, b1, W2, b2):
    raise NotImplementedError("write your pallas kernel here")


# speedup vs baseline: 1.3735x; 1.3735x over previous
import functools

import jax
import jax.numpy as jnp
from jax import lax
from jax.experimental import pallas as pl
from jax.experimental.pallas import tpu as pltpu
from jax.experimental.pallas import tpu_sc as plsc

N_EMBED = 1024
NUM_EXPERTS = 8
TOP_K = 2
HIDDEN = 4 * N_EMBED
N_TOKENS = 4096
N_ASSIGN = N_TOKENS * TOP_K

BLK = 256
NUM_BLOCKS = N_ASSIGN // BLK + NUM_EXPERTS
PADDED = NUM_BLOCKS * BLK

SC_CORES = 2
SC_SUBCORES = 16
NW = SC_CORES * SC_SUBCORES

G_ROWS_W = PADDED // NW
G_CH = 64
G_NCH = G_ROWS_W // G_CH

C_ROWS_W = N_TOKENS // NW
C_CH = 32
C_NCH = C_ROWS_W // C_CH

GATE_PAD = 128
GATE_ROWS = 512


def _gating_body(x_ref, gw_ref, gb_ref, i1_ref, i2_ref, wa_ref, wb_ref):
    logits = jnp.dot(x_ref[...], gw_ref[...],
                     preferred_element_type=jnp.float32) + gb_ref[...]
    col = lax.broadcasted_iota(jnp.int32, (GATE_ROWS, GATE_PAD), 1)
    m1 = jnp.max(logits, axis=1, keepdims=True)
    i1 = jnp.min(jnp.where(logits == m1, col, GATE_PAD), axis=1, keepdims=True)
    masked = jnp.where(col == i1, -jnp.inf, logits)
    m2 = jnp.max(masked, axis=1, keepdims=True)
    i2 = jnp.min(jnp.where(masked == m2, col, GATE_PAD), axis=1, keepdims=True)
    e2 = jnp.exp(m2 - m1)
    denom = 1.0 + e2
    i1_ref[...] = jnp.broadcast_to(i1, (GATE_ROWS, GATE_PAD))
    i2_ref[...] = jnp.broadcast_to(i2, (GATE_ROWS, GATE_PAD))
    wa_ref[...] = jnp.broadcast_to(1.0 / denom, (GATE_ROWS, GATE_PAD))
    wb_ref[...] = jnp.broadcast_to(e2 / denom, (GATE_ROWS, GATE_PAD))


def _gating_call(x_flat, gw_pad, gb_pad):
    n = x_flat.shape[0]
    grid = (n // GATE_ROWS,)
    out_shape = [
        jax.ShapeDtypeStruct((n, GATE_PAD), jnp.int32),
        jax.ShapeDtypeStruct((n, GATE_PAD), jnp.int32),
        jax.ShapeDtypeStruct((n, GATE_PAD), jnp.float32),
        jax.ShapeDtypeStruct((n, GATE_PAD), jnp.float32),
    ]
    spec_rows = pl.BlockSpec((GATE_ROWS, N_EMBED), lambda g: (g, 0))
    spec_out = pl.BlockSpec((GATE_ROWS, GATE_PAD), lambda g: (g, 0))
    return pl.pallas_call(
        _gating_body,
        grid=grid,
        in_specs=[
            spec_rows,
            pl.BlockSpec((N_EMBED, GATE_PAD), lambda g: (0, 0)),
            pl.BlockSpec((1, GATE_PAD), lambda g: (0, 0)),
        ],
        out_specs=[spec_out, spec_out, spec_out, spec_out],
        out_shape=out_shape,
    )(x_flat, gw_pad, gb_pad)


def _ffn1_body(be_ref, xg_ref, w1_ref, b1_ref, h_ref):
    xb = xg_ref[...].astype(jnp.bfloat16)
    w1 = w1_ref[0].astype(jnp.bfloat16)
    h = jnp.dot(xb, w1, preferred_element_type=jnp.float32)
    h_ref[...] = jnp.maximum(h + b1_ref[0], 0.0).astype(jnp.bfloat16)


def _ffn1_call(be, xg, W1, b1):
    grid_spec = pltpu.PrefetchScalarGridSpec(
        num_scalar_prefetch=1,
        grid=(NUM_BLOCKS,),
        in_specs=[
            pl.BlockSpec((BLK, N_EMBED), lambda g, be: (g, 0)),
            pl.BlockSpec((1, N_EMBED, HIDDEN), lambda g, be: (be[g], 0, 0)),
            pl.BlockSpec((1, 1, HIDDEN), lambda g, be: (be[g], 0, 0)),
        ],
        out_specs=pl.BlockSpec((BLK, HIDDEN), lambda g, be: (g, 0)),
    )
    return pl.pallas_call(
        _ffn1_body,
        grid_spec=grid_spec,
        out_shape=jax.ShapeDtypeStruct((PADDED, HIDDEN), jnp.bfloat16),
    )(be, xg, W1, b1)


def _ffn2_body(be_ref, h_ref, w2_ref, b2_ref, ws_ref, out_ref):
    w2 = w2_ref[0].astype(jnp.bfloat16)
    o = jnp.dot(h_ref[...], w2, preferred_element_type=jnp.float32)
    out_ref[...] = (o + b2_ref[0]) * ws_ref[...]


def _ffn2_call(be, h, W2, b2, ws):
    grid_spec = pltpu.PrefetchScalarGridSpec(
        num_scalar_prefetch=1,
        grid=(NUM_BLOCKS,),
        in_specs=[
            pl.BlockSpec((BLK, HIDDEN), lambda g, be: (g, 0)),
            pl.BlockSpec((1, HIDDEN, N_EMBED), lambda g, be: (be[g], 0, 0)),
            pl.BlockSpec((1, 1, N_EMBED), lambda g, be: (be[g], 0, 0)),
            pl.BlockSpec((BLK, 1), lambda g, be: (g, 0)),
        ],
        out_specs=pl.BlockSpec((BLK, N_EMBED), lambda g, be: (g, 0)),
    )
    return pl.pallas_call(
        _ffn2_body,
        grid_spec=grid_spec,
        out_shape=jax.ShapeDtypeStruct((PADDED, N_EMBED), jnp.float32),
    )(be, h, W2, b2, ws)


def _gather_body(x_hbm, idx_hbm, out_hbm, idx_v, rows_v, sem):
    wid = lax.axis_index("s") * SC_CORES + lax.axis_index("c")
    pltpu.sync_copy(idx_hbm.at[wid], idx_v)
    for c in range(G_NCH):
        pltpu.async_copy(x_hbm.at[idx_v.at[c]], rows_v, sem).wait()
        pltpu.sync_copy(rows_v,
                        out_hbm.at[pl.ds(wid * G_ROWS_W + c * G_CH, G_CH)])


def _gather_call(x_flat, idx3):
    mesh = plsc.VectorSubcoreMesh(core_axis_name="c", subcore_axis_name="s")
    f = functools.partial(
        pl.kernel,
        mesh=mesh,
        out_type=jax.ShapeDtypeStruct((PADDED, N_EMBED), jnp.float32),
        scratch_types=[
            pltpu.VMEM((G_NCH, G_CH), jnp.int32),
            pltpu.VMEM((G_CH, N_EMBED), jnp.float32),
            pltpu.SemaphoreType.DMA,
        ],
    )(_gather_body)
    return f(x_flat, idx3)


def _combine_body(os_hbm, pp_hbm, out_hbm, idx_v, buf_a, buf_b, sem_a, sem_b):
    wid = lax.axis_index("s") * SC_CORES + lax.axis_index("c")
    pltpu.sync_copy(pp_hbm.at[wid], idx_v)
    for c in range(C_NCH):
        cp_a = pltpu.async_copy(os_hbm.at[idx_v.at[c, 0]], buf_a, sem_a)
        cp_b = pltpu.async_copy(os_hbm.at[idx_v.at[c, 1]], buf_b, sem_b)
        cp_a.wait()
        cp_b.wait()

        def body(j, _):
            i = j // (N_EMBED // 16)
            off = (j % (N_EMBED // 16)) * 16
            a = buf_a[i, pl.ds(off, 16)]
            b = buf_b[i, pl.ds(off, 16)]
            buf_a[i, pl.ds(off, 16)] = a + b
            return 0

        lax.fori_loop(0, C_CH * (N_EMBED // 16), body, 0)
        pltpu.sync_copy(buf_a,
                        out_hbm.at[pl.ds(wid * C_ROWS_W + c * C_CH, C_CH)])


def _combine_call(out_sorted, pp):
    mesh = plsc.VectorSubcoreMesh(core_axis_name="c", subcore_axis_name="s")
    f = functools.partial(
        pl.kernel,
        mesh=mesh,
        out_type=jax.ShapeDtypeStruct((N_TOKENS, N_EMBED), jnp.float32),
        scratch_types=[
            pltpu.VMEM((C_NCH, 2, C_CH), jnp.int32),
            pltpu.VMEM((C_CH, N_EMBED), jnp.float32),
            pltpu.VMEM((C_CH, N_EMBED), jnp.float32),
            pltpu.SemaphoreType.DMA,
            pltpu.SemaphoreType.DMA,
        ],
    )(_combine_body)
    return f(out_sorted, pp)


def _routing_metadata(top2i, top2w):
    ef = top2i.reshape(-1)
    oh = (ef[:, None] == jnp.arange(NUM_EXPERTS, dtype=jnp.int32)[None, :])
    cum = jnp.cumsum(oh.astype(jnp.int32), axis=0)
    counts = cum[-1]
    rank = jnp.take_along_axis(cum, ef[:, None], axis=1)[:, 0] - 1
    pc = ((counts + BLK - 1) // BLK) * BLK
    cum_pc = jnp.cumsum(pc)
    po = cum_pc - pc
    dest = (po[ef] + rank).astype(jnp.int32)
    tok = (jnp.arange(N_ASSIGN, dtype=jnp.int32) // TOP_K)
    tok_sorted = jnp.zeros((PADDED,), jnp.int32).at[dest].set(tok)
    ws = jnp.zeros((PADDED,), jnp.float32).at[dest].set(top2w.reshape(-1))
    be = jnp.searchsorted(
        cum_pc, jnp.arange(NUM_BLOCKS, dtype=jnp.int32) * BLK, side='right')
    be = jnp.minimum(be, NUM_EXPERTS - 1).astype(jnp.int32)
    return dest, tok_sorted, ws, be


def kernel(x, gate_W, gate_b, W1, b1, W2, b2):
    b, t, c = x.shape
    x_flat = x.reshape(-1, c)

    gw_pad = jnp.zeros((N_EMBED, GATE_PAD), jnp.float32)
    gw_pad = lax.dynamic_update_slice(gw_pad, gate_W, (0, 0))
    gb_pad = jnp.full((1, GATE_PAD), -jnp.inf, jnp.float32)
    gb_pad = lax.dynamic_update_slice(gb_pad, gate_b[None, :], (0, 0))
    i1, i2, wa, wb = _gating_call(x_flat, gw_pad, gb_pad)
    top2i = jnp.stack([i1[:, 0], i2[:, 0]], axis=1)
    top2w = jnp.stack([wa[:, 0], wb[:, 0]], axis=1)

    dest, tok_sorted, ws, be = _routing_metadata(top2i, top2w)

    idx3 = tok_sorted.reshape(NW, G_NCH, G_CH)
    xg = _gather_call(x_flat, idx3)

    h = _ffn1_call(be, xg, W1, b1.reshape(NUM_EXPERTS, 1, HIDDEN))
    out_sorted = _ffn2_call(be, h, W2, b2.reshape(NUM_EXPERTS, 1, N_EMBED),
                            ws[:, None])

    pp = dest.reshape(N_TOKENS, TOP_K).reshape(NW, C_NCH, C_CH, TOP_K)
    pp = jnp.transpose(pp, (0, 1, 3, 2))
    final = _combine_call(out_sorted, pp)

    return final.reshape(b, t, c)
